# baseline (speedup 1.0000x reference)
import functools

import jax
import jax.numpy as jnp
from jax import lax
from jax.experimental import pallas as pl
from jax.experimental.pallas import tpu as pltpu
from jax.experimental.pallas import tpu_sc as plsc

N_NODES = 10000
D = 128
N_EDGES = 320000

NC = 2
NS = 16
NW = NC * NS

CHUNK = 128
N_BLOCKS = 5
BLK_CH = 16
CH_PER_W = N_BLOCKS * BLK_CH
E_PAD = NW * CH_PER_W * CHUNK
N_PAD = 10112
ROWS_PER_TILE = N_PAD // NS


def _sc_agg_body(x_hbm, row_hbm, col_hbm, zeros_hbm, out_hbm,
                 col_v, row_v, buf0, buf1, agg_sh, sem0, sem1):
    cid = lax.axis_index("c")
    sid = lax.axis_index("s")
    wid = sid * NC + cid

    pltpu.sync_copy(zeros_hbm,
                    agg_sh.at[pl.ds(sid * ROWS_PER_TILE, ROWS_PER_TILE)])
    plsc.subcore_barrier()

    def outer(b, carry):
        pltpu.sync_copy(row_hbm.at[wid, b], row_v)
        pltpu.sync_copy(col_hbm.at[wid, b], col_v)

        def body(a, carry2):
            pltpu.async_copy(x_hbm.at[col_v.at[a]], buf0, sem0).wait()
            pltpu.sync_copy(buf0, agg_sh.at[row_v.at[a]], add=True)
            return carry2

        lax.fori_loop(0, BLK_CH, body, 0)
        return carry

    lax.fori_loop(0, N_BLOCKS, outer, 0)
    plsc.subcore_barrier()

    sl = pl.ds(sid * ROWS_PER_TILE, ROWS_PER_TILE)
    pltpu.sync_copy(agg_sh.at[sl], out_hbm.at[cid, sl])


_sc_agg = pl.kernel(
    _sc_agg_body,
    out_type=jax.ShapeDtypeStruct((NC, N_PAD, D), jnp.float32),
    mesh=plsc.VectorSubcoreMesh(core_axis_name="c", subcore_axis_name="s"),
    scratch_types=[
        pltpu.VMEM((BLK_CH, CHUNK), jnp.int32),
        pltpu.VMEM((BLK_CH, CHUNK), jnp.int32),
        pltpu.VMEM((CHUNK, D), jnp.float32),
        pltpu.VMEM((CHUNK, D), jnp.float32),
        pltpu.VMEM_SHARED((N_PAD, D), jnp.float32),
        pltpu.SemaphoreType.DMA,
        pltpu.SemaphoreType.DMA,
    ],
)


def _mlp_body(eps_sr, x_r, p0_r, p1_r, w1_r, b1_r, w2_r, b2_r, o_r):
    scale = 1.0 + eps_sr[0]
    h = scale * x_r[...] + (p0_r[0] + p1_r[0])
    h = jnp.dot(h, w1_r[...], preferred_element_type=jnp.float32) + b1_r[...]
    h = jnp.maximum(h, 0.0)
    o_r[...] = (jnp.dot(h, w2_r[...], preferred_element_type=jnp.float32)
                + b2_r[...])


BLK = 1000


def kernel(x, edge_index, num_nodes, eps, W1, b1, W2, b2):
    row = edge_index[0].astype(jnp.int32)
    col = edge_index[1].astype(jnp.int32)
    pad = E_PAD - N_EDGES
    row_p = jnp.concatenate(
        [row, jnp.full((pad,), N_NODES, jnp.int32)]
    ).reshape(NW, N_BLOCKS, BLK_CH, CHUNK)
    col_p = jnp.concatenate(
        [col, jnp.zeros((pad,), jnp.int32)]
    ).reshape(NW, N_BLOCKS, BLK_CH, CHUNK)
    zeros = jnp.zeros((ROWS_PER_TILE, D), jnp.float32)

    partials = _sc_agg(x, row_p, col_p, zeros)

    grid = (x.shape[0] // BLK,)
    out = pl.pallas_call(
        _mlp_body,
        grid=grid,
        in_specs=[
            pl.BlockSpec(memory_space=pltpu.SMEM),
            pl.BlockSpec((BLK, D), lambda i: (i, 0)),
            pl.BlockSpec((1, BLK, D), lambda i: (0, i, 0)),
            pl.BlockSpec((1, BLK, D), lambda i: (1, i, 0)),
            pl.BlockSpec((D, D), lambda i: (0, 0)),
            pl.BlockSpec((1, D), lambda i: (0, 0)),
            pl.BlockSpec((D, D), lambda i: (0, 0)),
            pl.BlockSpec((1, D), lambda i: (0, 0)),
        ],
        out_specs=pl.BlockSpec((BLK, D), lambda i: (i, 0)),
        out_shape=jax.ShapeDtypeStruct((x.shape[0], D), jnp.float32),
    )(eps, x, partials, partials, W1, b1.reshape(1, D), W2, b2.reshape(1, D))
    return out


# speedup vs baseline: 1.5081x; 1.5081x over previous
import jax
import jax.numpy as jnp
from jax import lax
from jax.experimental import pallas as pl
from jax.experimental.pallas import tpu as pltpu
from jax.experimental.pallas import tpu_sc as plsc

N_NODES = 10000
D = 128
N_EDGES = 320000

NC = 2
NS = 16
NW = NC * NS

CHUNK = 128
CH_PER_W = 79
E_PAD = NW * CH_PER_W * CHUNK
N_PAD = 10112
ROWS_PER_TILE = N_PAD // NS


def _sc_agg_body(x_hbm, row_hbm, col_hbm, zeros_hbm, out_hbm,
                 col_v, row_v, buf, agg_sh, sem):
    cid = lax.axis_index("c")
    sid = lax.axis_index("s")
    wid = sid * NC + cid

    pltpu.sync_copy(zeros_hbm,
                    agg_sh.at[pl.ds(sid * ROWS_PER_TILE, ROWS_PER_TILE)])
    pltpu.sync_copy(row_hbm.at[wid], row_v)
    pltpu.sync_copy(col_hbm.at[wid], col_v)
    plsc.subcore_barrier()

    def body(c, carry):
        pltpu.async_copy(x_hbm.at[col_v.at[c]], buf, sem).wait()
        pltpu.sync_copy(buf, agg_sh.at[row_v.at[c]], add=True)
        return carry

    lax.fori_loop(0, CH_PER_W, body, 0)
    plsc.subcore_barrier()

    sl = pl.ds(sid * ROWS_PER_TILE, ROWS_PER_TILE)
    pltpu.sync_copy(agg_sh.at[sl], out_hbm.at[cid, sl])


_sc_agg = pl.kernel(
    _sc_agg_body,
    out_type=jax.ShapeDtypeStruct((NC, N_PAD, D), jnp.float32),
    mesh=plsc.VectorSubcoreMesh(core_axis_name="c", subcore_axis_name="s"),
    scratch_types=[
        pltpu.VMEM((CH_PER_W, CHUNK), jnp.int32),
        pltpu.VMEM((CH_PER_W, CHUNK), jnp.int32),
        pltpu.VMEM((CHUNK, D), jnp.float32),
        pltpu.VMEM_SHARED((N_PAD, D), jnp.float32),
        pltpu.SemaphoreType.DMA,
    ],
)


def _mlp_body(eps_sr, x_r, p0_r, p1_r, w1_r, b1_r, w2_r, b2_r, o_r):
    scale = 1.0 + eps_sr[0]
    h = scale * x_r[...] + (p0_r[0] + p1_r[0])
    h = jnp.dot(h, w1_r[...], preferred_element_type=jnp.float32) + b1_r[...]
    h = jnp.maximum(h, 0.0)
    o_r[...] = (jnp.dot(h, w2_r[...], preferred_element_type=jnp.float32)
                + b2_r[...])


BLK = 1000


def kernel(x, edge_index, num_nodes, eps, W1, b1, W2, b2):
    row = edge_index[0].astype(jnp.int32)
    col = edge_index[1].astype(jnp.int32)
    pad = E_PAD - N_EDGES
    row_p = jnp.concatenate(
        [row, jnp.full((pad,), N_NODES, jnp.int32)]).reshape(NW, CH_PER_W, CHUNK)
    col_p = jnp.concatenate(
        [col, jnp.zeros((pad,), jnp.int32)]).reshape(NW, CH_PER_W, CHUNK)
    zeros = jnp.zeros((ROWS_PER_TILE, D), jnp.float32)

    partials = _sc_agg(x, row_p, col_p, zeros)

    grid = (x.shape[0] // BLK,)
    out = pl.pallas_call(
        _mlp_body,
        grid=grid,
        in_specs=[
            pl.BlockSpec(memory_space=pltpu.SMEM),
            pl.BlockSpec((BLK, D), lambda i: (i, 0)),
            pl.BlockSpec((1, BLK, D), lambda i: (0, i, 0)),
            pl.BlockSpec((1, BLK, D), lambda i: (1, i, 0)),
            pl.BlockSpec((D, D), lambda i: (0, 0)),
            pl.BlockSpec((1, D), lambda i: (0, 0)),
            pl.BlockSpec((D, D), lambda i: (0, 0)),
            pl.BlockSpec((1, D), lambda i: (0, 0)),
        ],
        out_specs=pl.BlockSpec((BLK, D), lambda i: (i, 0)),
        out_shape=jax.ShapeDtypeStruct((x.shape[0], D), jnp.float32),
    )(eps, x, partials, partials, W1, b1.reshape(1, D), W2, b2.reshape(1, D))
    return out
